# Initial kernel scaffold; baseline (speedup 1.0000x reference)
import jax
import jax.numpy as jnp
from jax.experimental import pallas as pl


def kernel(outputs, targets):
    raise NotImplementedError("write your pallas kernel here")


# speedup vs baseline: 1.1779x; 1.1779x over previous
import functools

import jax
import jax.numpy as jnp
from jax import lax
from jax.experimental import pallas as pl
from jax.experimental.pallas import tpu as pltpu
from jax.experimental.pallas import tpu_sc as plsc

B = 1024
N = 100000
CB = 2048
NB = (N + CB - 1) // CB


def _sc_gather(outputs_flat, targets_i32):
    info = plsc.get_sparse_core_info()
    nc, ns, lanes = info.num_cores, info.num_subcores, info.num_lanes
    nw = nc * ns
    bpw = B // nw

    mesh = plsc.VectorSubcoreMesh(core_axis_name="c", subcore_axis_name="s")

    @functools.partial(
        pl.kernel,
        out_type=jax.ShapeDtypeStruct((B,), jnp.float32),
        mesh=mesh,
        scratch_types=[
            pltpu.VMEM((bpw,), jnp.int32),
            pltpu.VMEM((bpw,), jnp.int32),
            pltpu.VMEM((bpw,), jnp.float32),
            pltpu.SemaphoreType.DMA,
        ],
    )
    def gather_kernel(out_hbm, tgt_hbm, vals_hbm, tgt_v, flat_v, vals_v, sem):
        wid = lax.axis_index("s") * nc + lax.axis_index("c")
        base = wid * bpw
        pltpu.sync_copy(tgt_hbm.at[pl.ds(base, bpw)], tgt_v)
        for c in range(bpw // lanes):
            t16 = tgt_v[pl.ds(c * lanes, lanes)]
            row = lax.iota(jnp.int32, lanes) + (base + c * lanes)
            flat_v[pl.ds(c * lanes, lanes)] = row * N + t16
        pltpu.async_copy(out_hbm.at[flat_v], vals_v, sem).wait()
        pltpu.sync_copy(vals_v, vals_hbm.at[pl.ds(base, bpw)])

    return gather_kernel(outputs_flat, targets_i32)


def _count_body(x_ref, v_ref, t_ref, o1_ref, o5_ref, acc_ref):
    j = pl.program_id(0)
    x = x_ref[...]
    v = v_ref[...]
    t = t_ref[...]
    col = lax.broadcasted_iota(jnp.int32, (B, CB), 1) + j * CB
    beats = (x == v) & (col < t)

    @pl.when(j < NB - 1)
    def _():
        cnt = jnp.sum(((x > v) | beats).astype(jnp.int32), axis=1,
                      keepdims=True)
        if NB == 1:
            acc_ref[...] = cnt
        else:
            @pl.when(j == 0)
            def _():
                acc_ref[...] = cnt

            @pl.when(j > 0)
            def _():
                acc_ref[...] = acc_ref[...] + cnt

    @pl.when(j == NB - 1)
    def _():
        gt = (x > v) & (col < N)
        cnt = jnp.sum((gt | beats).astype(jnp.int32), axis=1, keepdims=True)
        if NB == 1:
            rank = cnt
        else:
            rank = acc_ref[...] + cnt
        scale = jnp.float32(100.0 / B)
        o1_ref[...] = jnp.sum((rank < 1).astype(jnp.float32), axis=0,
                              keepdims=True) * scale
        o5_ref[...] = jnp.sum((rank < 5).astype(jnp.float32), axis=0,
                              keepdims=True) * scale


def _tc_count(outputs, v_col, t_col, interpret=False):
    out1, out5 = pl.pallas_call(
        _count_body,
        grid=(NB,),
        in_specs=[
            pl.BlockSpec((B, CB), lambda j: (0, j)),
            pl.BlockSpec((B, 1), lambda j: (0, 0)),
            pl.BlockSpec((B, 1), lambda j: (0, 0)),
        ],
        out_specs=[
            pl.BlockSpec((1, 1), lambda j: (0, 0)),
            pl.BlockSpec((1, 1), lambda j: (0, 0)),
        ],
        out_shape=[
            jax.ShapeDtypeStruct((1, 1), jnp.float32),
            jax.ShapeDtypeStruct((1, 1), jnp.float32),
        ],
        scratch_shapes=[pltpu.VMEM((B, 1), jnp.int32)],
        interpret=interpret,
    )(outputs, v_col, t_col)
    return out1, out5


def kernel(outputs, targets):
    t32 = targets.astype(jnp.int32)
    v = _sc_gather(outputs.reshape(-1), t32)
    out1, out5 = _tc_count(outputs, v.reshape(B, 1), t32.reshape(B, 1))
    return (out1.reshape(1), out5.reshape(1))


# speedup vs baseline: 2.1617x; 1.8352x over previous
import functools

import jax
import jax.numpy as jnp
from jax import lax
from jax.experimental import pallas as pl
from jax.experimental.pallas import tpu as pltpu

B = 1024
N = 100000
CB = 2048
NB = (N + CB - 1) // CB
GR = 8


def _gather_body(t_ref, *refs):
    out_ref = refs[-1]
    j = pl.program_id(0)
    lane = lax.broadcasted_iota(jnp.int32, (1, 128), 1)
    for m in range(GR):
        t = t_ref[j * GR + m]
        l = lax.rem(t, 128)
        xm = refs[m][pl.ds(m, 1), :]
        vm = jnp.sum(jnp.where(lane == l, xm, 0.0), axis=1, keepdims=True)
        out_ref[pl.ds(m, 1), :] = vm


def _row_imap(m, j, t_ref):
    return (j, t_ref[j * GR + m] // 128)


def _gather(outputs, t32):
    grid_spec = pltpu.PrefetchScalarGridSpec(
        num_scalar_prefetch=1,
        grid=(B // GR,),
        in_specs=[
            pl.BlockSpec((GR, 128), functools.partial(_row_imap, m))
            for m in range(GR)
        ],
        out_specs=pl.BlockSpec((GR, 1), lambda j, t_ref: (j, 0)),
    )
    return pl.pallas_call(
        _gather_body,
        grid_spec=grid_spec,
        out_shape=jax.ShapeDtypeStruct((B, 1), jnp.float32),
    )(t32, *([outputs] * GR))


def _count_body(x_ref, v_ref, t_ref, o1_ref, o5_ref, acc_ref):
    j = pl.program_id(0)
    x = x_ref[...]
    v = v_ref[...]
    t = t_ref[...]
    col = lax.broadcasted_iota(jnp.int32, (B, CB), 1) + j * CB
    beats = ((x > v) & (col < N)) | ((x == v) & (col < t))
    cnt = jnp.sum(beats.astype(jnp.int32), axis=1, keepdims=True)

    @pl.when(j == 0)
    def _():
        acc_ref[...] = cnt

    @pl.when(j > 0)
    def _():
        acc_ref[...] = acc_ref[...] + cnt

    @pl.when(j == NB - 1)
    def _():
        rank = acc_ref[...]
        scale = jnp.float32(100.0 / B)
        o1_ref[...] = jnp.sum((rank < 1).astype(jnp.float32), axis=0,
                              keepdims=True) * scale
        o5_ref[...] = jnp.sum((rank < 5).astype(jnp.float32), axis=0,
                              keepdims=True) * scale


def _tc_count(outputs, v_col, t_col, interpret=False):
    out1, out5 = pl.pallas_call(
        _count_body,
        grid=(NB,),
        in_specs=[
            pl.BlockSpec((B, CB), lambda j: (0, j)),
            pl.BlockSpec((B, 1), lambda j: (0, 0)),
            pl.BlockSpec((B, 1), lambda j: (0, 0)),
        ],
        out_specs=[
            pl.BlockSpec((1, 1), lambda j: (0, 0)),
            pl.BlockSpec((1, 1), lambda j: (0, 0)),
        ],
        out_shape=[
            jax.ShapeDtypeStruct((1, 1), jnp.float32),
            jax.ShapeDtypeStruct((1, 1), jnp.float32),
        ],
        scratch_shapes=[pltpu.VMEM((B, 1), jnp.int32)],
        interpret=interpret,
    )(outputs, v_col, t_col)
    return out1, out5


def kernel(outputs, targets):
    t32 = targets.astype(jnp.int32)
    v = _gather(outputs, t32)
    out1, out5 = _tc_count(outputs, v, t32.reshape(B, 1))
    return (out1.reshape(1), out5.reshape(1))


# speedup vs baseline: 4.4403x; 2.0540x over previous
import functools

import jax
import jax.numpy as jnp
from jax import lax
from jax.experimental import pallas as pl
from jax.experimental.pallas import tpu as pltpu

B = 1024
N = 100000
VB = 2048
NB = (N + VB - 1) // VB
GR = 8


def _gather_body(t_ref, *refs):
    out_ref = refs[-1]
    j = pl.program_id(0)
    sub = lax.broadcasted_iota(jnp.int32, (GR, 128), 0)
    lane = lax.broadcasted_iota(jnp.int32, (GR, 128), 1)
    for m in range(GR):
        r = j * GR + m
        t = t_ref[r]
        hit = (sub == lax.rem(t, GR)) & (lane == lax.rem(r, 128))
        x = refs[m][...]
        vm = jnp.sum(jnp.where(hit, x, 0.0), axis=(0, 1), keepdims=True)
        out_ref[pl.ds(m, 1), :] = vm


def _tile_imap(m, j, t_ref):
    r = j * GR + m
    return (t_ref[r] // GR, r // 128)


def _gather(outputs_t, t32):
    grid_spec = pltpu.PrefetchScalarGridSpec(
        num_scalar_prefetch=1,
        grid=(B // GR,),
        in_specs=[
            pl.BlockSpec((GR, 128), functools.partial(_tile_imap, m))
            for m in range(GR)
        ],
        out_specs=pl.BlockSpec((GR, 1), lambda j, t_ref: (j, 0)),
    )
    return pl.pallas_call(
        _gather_body,
        grid_spec=grid_spec,
        out_shape=jax.ShapeDtypeStruct((B, 1), jnp.float32),
    )(t32, *([outputs_t] * GR))


def _count_body(x_ref, v_ref, t_ref, o1_ref, o5_ref, acc_ref):
    j = pl.program_id(0)
    x = x_ref[...]
    v = v_ref[...]
    t = t_ref[...]
    col = lax.broadcasted_iota(jnp.int32, (VB, B), 0) + j * VB
    beats = ((x > v) & (col < N)) | ((x == v) & (col < t))
    cnt = jnp.sum(beats.astype(jnp.int32), axis=0, keepdims=True)

    @pl.when(j == 0)
    def _():
        acc_ref[...] = cnt

    @pl.when(j > 0)
    def _():
        acc_ref[...] = acc_ref[...] + cnt

    @pl.when(j == NB - 1)
    def _():
        rank = acc_ref[...]
        scale = jnp.float32(100.0 / B)
        o1_ref[...] = jnp.sum((rank < 1).astype(jnp.float32), axis=1,
                              keepdims=True) * scale
        o5_ref[...] = jnp.sum((rank < 5).astype(jnp.float32), axis=1,
                              keepdims=True) * scale


def _tc_count(outputs_t, v_row, t_row, interpret=False):
    out1, out5 = pl.pallas_call(
        _count_body,
        grid=(NB,),
        in_specs=[
            pl.BlockSpec((VB, B), lambda j: (j, 0)),
            pl.BlockSpec((1, B), lambda j: (0, 0)),
            pl.BlockSpec((1, B), lambda j: (0, 0)),
        ],
        out_specs=[
            pl.BlockSpec((1, 1), lambda j: (0, 0)),
            pl.BlockSpec((1, 1), lambda j: (0, 0)),
        ],
        out_shape=[
            jax.ShapeDtypeStruct((1, 1), jnp.float32),
            jax.ShapeDtypeStruct((1, 1), jnp.float32),
        ],
        scratch_shapes=[pltpu.VMEM((1, B), jnp.int32)],
        interpret=interpret,
    )(outputs_t, v_row, t_row)
    return out1, out5


def kernel(outputs, targets):
    t32 = targets.astype(jnp.int32)
    xt = outputs.T
    v = _gather(xt, t32)
    out1, out5 = _tc_count(xt, v.reshape(1, B), t32.reshape(1, B))
    return (out1.reshape(1), out5.reshape(1))


# speedup vs baseline: 4.7876x; 1.0782x over previous
import functools

import jax
import jax.numpy as jnp
from jax import lax
from jax.experimental import pallas as pl
from jax.experimental.pallas import tpu as pltpu

B = 1024
N = 100000
VB = 4096
NB = (N + VB - 1) // VB
GR = 32


def _gather_body(t_ref, *refs):
    out_ref = refs[-1]
    j = pl.program_id(0)
    sub = lax.broadcasted_iota(jnp.int32, (8, 128), 0)
    lane = lax.broadcasted_iota(jnp.int32, (8, 128), 1)
    for m in range(GR):
        r = j * GR + m
        t = t_ref[r]
        hit = (sub == lax.rem(t, 8)) & (lane == lax.rem(r, 128))
        x = refs[m][...]
        vm = jnp.sum(jnp.where(hit, x, 0.0), axis=(0, 1), keepdims=True)
        out_ref[pl.ds(m, 1), :] = vm


def _tile_imap(m, j, t_ref):
    r = j * GR + m
    return (t_ref[r] // 8, r // 128)


def _gather(outputs_t, t32):
    grid_spec = pltpu.PrefetchScalarGridSpec(
        num_scalar_prefetch=1,
        grid=(B // GR,),
        in_specs=[
            pl.BlockSpec((8, 128), functools.partial(_tile_imap, m))
            for m in range(GR)
        ],
        out_specs=pl.BlockSpec((GR, 1), lambda j, t_ref: (j, 0)),
    )
    return pl.pallas_call(
        _gather_body,
        grid_spec=grid_spec,
        out_shape=jax.ShapeDtypeStruct((B, 1), jnp.float32),
    )(t32, *([outputs_t] * GR))


def _count_body(x_ref, v_ref, t_ref, o1_ref, o5_ref, acc_ref):
    j = pl.program_id(0)
    x = x_ref[...]
    v = v_ref[...]
    t = t_ref[...]
    col = lax.broadcasted_iota(jnp.int32, (VB, B), 0) + j * VB
    beats = ((x > v) & (col < N)) | ((x == v) & (col < t))
    cnt = jnp.sum(beats.astype(jnp.int32), axis=0, keepdims=True)

    @pl.when(j == 0)
    def _():
        acc_ref[...] = cnt

    @pl.when(j > 0)
    def _():
        acc_ref[...] = acc_ref[...] + cnt

    @pl.when(j == NB - 1)
    def _():
        rank = acc_ref[...]
        scale = jnp.float32(100.0 / B)
        o1_ref[...] = jnp.sum((rank < 1).astype(jnp.float32), axis=1,
                              keepdims=True) * scale
        o5_ref[...] = jnp.sum((rank < 5).astype(jnp.float32), axis=1,
                              keepdims=True) * scale


def _tc_count(outputs_t, v_row, t_row, interpret=False):
    out1, out5 = pl.pallas_call(
        _count_body,
        grid=(NB,),
        in_specs=[
            pl.BlockSpec((VB, B), lambda j: (j, 0)),
            pl.BlockSpec((1, B), lambda j: (0, 0)),
            pl.BlockSpec((1, B), lambda j: (0, 0)),
        ],
        out_specs=[
            pl.BlockSpec((1, 1), lambda j: (0, 0)),
            pl.BlockSpec((1, 1), lambda j: (0, 0)),
        ],
        out_shape=[
            jax.ShapeDtypeStruct((1, 1), jnp.float32),
            jax.ShapeDtypeStruct((1, 1), jnp.float32),
        ],
        scratch_shapes=[pltpu.VMEM((1, B), jnp.int32)],
        interpret=interpret,
    )(outputs_t, v_row, t_row)
    return out1, out5


def kernel(outputs, targets):
    t32 = targets.astype(jnp.int32)
    xt = outputs.T
    v = _gather(xt, t32)
    out1, out5 = _tc_count(xt, v.reshape(1, B), t32.reshape(1, B))
    return (out1.reshape(1), out5.reshape(1))


# speedup vs baseline: 6.0726x; 1.2684x over previous
import functools

import jax
import jax.numpy as jnp
from jax import lax
from jax.experimental import pallas as pl
from jax.experimental.pallas import tpu as pltpu

B = 1024
N = 100000
VB = 4096
NB = (N + VB - 1) // VB
GR = 64


def _gather_body(t_ref, *refs):
    out_ref = refs[-1]
    j = pl.program_id(0)
    sub = lax.broadcasted_iota(jnp.int32, (8, 128), 0)
    lane = lax.broadcasted_iota(jnp.int32, (8, 128), 1)
    for m in range(GR):
        r = j * GR + m
        t = t_ref[r]
        hit = (sub == lax.rem(t, 8)) & (lane == lax.rem(r, 128))
        x = refs[m][...]
        vm = jnp.sum(jnp.where(hit, x, 0.0), axis=(0, 1), keepdims=True)
        out_ref[pl.ds(m, 1), :] = vm


def _tile_imap(m, j, t_ref):
    r = j * GR + m
    return (t_ref[r] // 8, r // 128)


def _gather(outputs_t, t32):
    grid_spec = pltpu.PrefetchScalarGridSpec(
        num_scalar_prefetch=1,
        grid=(B // GR,),
        in_specs=[
            pl.BlockSpec((8, 128), functools.partial(_tile_imap, m))
            for m in range(GR)
        ],
        out_specs=pl.BlockSpec((GR, 1), lambda j, t_ref: (j, 0)),
    )
    return pl.pallas_call(
        _gather_body,
        grid_spec=grid_spec,
        out_shape=jax.ShapeDtypeStruct((B, 1), jnp.float32),
    )(t32, *([outputs_t] * GR))


def _count_body(x_ref, v_ref, t_ref, o1_ref, o5_ref, acc_ref):
    j = pl.program_id(0)

    pad = NB * VB - N
    if pad:
        @pl.when(j == NB - 1)
        def _():
            x_ref[pl.ds(VB - pad, pad), :] = jnp.full(
                (pad, B), -jnp.inf, jnp.float32)

    x = x_ref[...]
    v = v_ref[...]
    t = t_ref[...]
    col = lax.broadcasted_iota(jnp.int32, (VB, B), 0) + j * VB
    beats = (x > v) | ((x == v) & (col < t))
    cnt = jnp.sum(beats.astype(jnp.int32), axis=0, keepdims=True)

    @pl.when(j == 0)
    def _():
        acc_ref[...] = cnt

    @pl.when(j > 0)
    def _():
        acc_ref[...] = acc_ref[...] + cnt

    @pl.when(j == NB - 1)
    def _():
        rank = acc_ref[...]
        scale = jnp.float32(100.0 / B)
        o1_ref[...] = jnp.sum((rank < 1).astype(jnp.float32), axis=1,
                              keepdims=True) * scale
        o5_ref[...] = jnp.sum((rank < 5).astype(jnp.float32), axis=1,
                              keepdims=True) * scale


def _tc_count(outputs_t, v_row, t_row, interpret=False):
    out1, out5 = pl.pallas_call(
        _count_body,
        grid=(NB,),
        in_specs=[
            pl.BlockSpec((VB, B), lambda j: (j, 0)),
            pl.BlockSpec((1, B), lambda j: (0, 0)),
            pl.BlockSpec((1, B), lambda j: (0, 0)),
        ],
        out_specs=[
            pl.BlockSpec((1, 1), lambda j: (0, 0)),
            pl.BlockSpec((1, 1), lambda j: (0, 0)),
        ],
        out_shape=[
            jax.ShapeDtypeStruct((1, 1), jnp.float32),
            jax.ShapeDtypeStruct((1, 1), jnp.float32),
        ],
        scratch_shapes=[pltpu.VMEM((1, B), jnp.int32)],
        interpret=interpret,
    )(outputs_t, v_row, t_row)
    return out1, out5


def kernel(outputs, targets):
    t32 = targets.astype(jnp.int32)
    xt = outputs.T
    v = _gather(xt, t32)
    out1, out5 = _tc_count(xt, v.reshape(1, B), t32.reshape(1, B))
    return (out1.reshape(1), out5.reshape(1))


# speedup vs baseline: 6.1214x; 1.0080x over previous
import functools

import jax
import jax.numpy as jnp
from jax import lax
from jax.experimental import pallas as pl
from jax.experimental.pallas import tpu as pltpu

B = 1024
N = 100000
VB = 4096
NB = (N + VB - 1) // VB
GR = 64
CH = 16
UNROLL = 4


def _gather_body(t_ref, *refs):
    out_ref = refs[-1]
    j = pl.program_id(0)
    sub = lax.broadcasted_iota(jnp.int32, (8, 128), 0)
    lane = lax.broadcasted_iota(jnp.int32, (8, 128), 1)
    for m in range(GR):
        r = j * GR + m
        t = t_ref[r]
        hit = (sub == lax.rem(t, 8)) & (lane == lax.rem(r, 128))
        x = refs[m][...]
        vm = jnp.sum(jnp.where(hit, x, 0.0), axis=(0, 1), keepdims=True)
        out_ref[pl.ds(m, 1), :] = vm


def _tile_imap(m, j, t_ref):
    r = j * GR + m
    return (t_ref[r] // 8, r // 128)


def _gather(outputs_t, t32):
    grid_spec = pltpu.PrefetchScalarGridSpec(
        num_scalar_prefetch=1,
        grid=(B // GR,),
        in_specs=[
            pl.BlockSpec((8, 128), functools.partial(_tile_imap, m))
            for m in range(GR)
        ],
        out_specs=pl.BlockSpec((GR, 1), lambda j, t_ref: (j, 0)),
    )
    return pl.pallas_call(
        _gather_body,
        grid_spec=grid_spec,
        out_shape=jax.ShapeDtypeStruct((B, 1), jnp.float32),
    )(t32, *([outputs_t] * GR))


def _count_body(x_ref, v_ref, t_ref, o1_ref, o5_ref, acc_ref):
    j = pl.program_id(0)

    pad = NB * VB - N
    if pad:
        @pl.when(j == NB - 1)
        def _():
            x_ref[pl.ds(VB - pad, pad), :] = jnp.full(
                (pad, B), -jnp.inf, jnp.float32)

    v = v_ref[...]
    t = t_ref[...]
    tl = t - j * VB
    iota = lax.broadcasted_iota(jnp.int32, (CH, B), 0)

    def chunk(c, acc):
        x = x_ref[pl.ds(c * CH, CH), :]
        sub = iota + c * CH
        beats = (x > v) | ((x == v) & (sub < tl))
        return acc + beats.astype(jnp.int32)

    psum = lax.fori_loop(0, VB // CH, chunk, jnp.zeros((CH, B), jnp.int32),
                         unroll=UNROLL)
    cnt = jnp.sum(psum, axis=0, keepdims=True)

    @pl.when(j == 0)
    def _():
        acc_ref[...] = cnt

    @pl.when(j > 0)
    def _():
        acc_ref[...] = acc_ref[...] + cnt

    @pl.when(j == NB - 1)
    def _():
        rank = acc_ref[...]
        scale = jnp.float32(100.0 / B)
        o1_ref[...] = jnp.sum((rank < 1).astype(jnp.float32), axis=1,
                              keepdims=True) * scale
        o5_ref[...] = jnp.sum((rank < 5).astype(jnp.float32), axis=1,
                              keepdims=True) * scale


def _tc_count(outputs_t, v_row, t_row, interpret=False):
    out1, out5 = pl.pallas_call(
        _count_body,
        grid=(NB,),
        in_specs=[
            pl.BlockSpec((VB, B), lambda j: (j, 0)),
            pl.BlockSpec((1, B), lambda j: (0, 0)),
            pl.BlockSpec((1, B), lambda j: (0, 0)),
        ],
        out_specs=[
            pl.BlockSpec((1, 1), lambda j: (0, 0)),
            pl.BlockSpec((1, 1), lambda j: (0, 0)),
        ],
        out_shape=[
            jax.ShapeDtypeStruct((1, 1), jnp.float32),
            jax.ShapeDtypeStruct((1, 1), jnp.float32),
        ],
        scratch_shapes=[pltpu.VMEM((1, B), jnp.int32)],
        interpret=interpret,
    )(outputs_t, v_row, t_row)
    return out1, out5


def kernel(outputs, targets):
    t32 = targets.astype(jnp.int32)
    xt = outputs.T
    v = _gather(xt, t32)
    out1, out5 = _tc_count(xt, v.reshape(1, B), t32.reshape(1, B))
    return (out1.reshape(1), out5.reshape(1))
